# baseline (speedup 1.0000x reference)
import jax
import jax.numpy as jnp
from jax import lax
from jax.experimental import pallas as pl
from jax.experimental.pallas import tpu as pltpu
from jax.experimental.pallas import tpu_sc as plsc

N_USERS = 6000
N_ITEMS = 4000
N_NODES = N_USERS + N_ITEMS
N_EDGES = 320000
DIM = 128
HALF = DIM // 2

N_CORES = 2
N_SUBCORES = 16
CHUNK = 128
SUBC = 8
SUPER = 10
EDGES_PER_TILE = SUPER * SUBC * CHUNK
N_EDGES_PAD = N_CORES * N_SUBCORES * EDGES_PER_TILE
ROWS_PER_TILE = 624
TAIL_ROWS = N_NODES - N_SUBCORES * ROWS_PER_TILE
ZROWS = 48
SUB = 104


def _scale_chunk(buf, valv, ci):
    def group(g, carry):
        vv = valv[ci, pl.ds(g * 16, 16)]
        for e in range(16):
            idx = jnp.full((16, 1), e, jnp.int32)
            vb = lax.gather(
                vv, idx,
                lax.GatherDimensionNumbers(offset_dims=(),
                                           collapsed_slice_dims=(0,),
                                           start_index_map=(0,)),
                slice_sizes=(1,),
                mode=lax.GatherScatterMode.PROMISE_IN_BOUNDS)
            r = g * 16 + e
            for j in range(HALF // 16):
                buf[r, pl.ds(16 * j, 16)] = buf[r, pl.ds(16 * j, 16)] * vb
        return carry

    lax.fori_loop(0, CHUNK // 16, group, 0)


def _add_rows(buf0, buf1, n):
    def addrow(r, carry):
        for j in range(HALF // 16):
            buf0[r, pl.ds(16 * j, 16)] = (buf0[r, pl.ds(16 * j, 16)] +
                                          buf1[r, pl.ds(16 * j, 16)])
        return carry

    lax.fori_loop(0, n, addrow, 0)


def _zero_acc(acc, zbuf, t, semz):
    zero = jnp.zeros((16,), jnp.float32)
    for r in range(ZROWS):
        for j in range(HALF // 16):
            zbuf[r, pl.ds(16 * j, 16)] = zero

    def zcopy(k, carry):
        pltpu.async_copy(
            zbuf, acc.at[pl.ds(t * ROWS_PER_TILE + k * ZROWS, ZROWS)], semz)
        return carry

    lax.fori_loop(0, ROWS_PER_TILE // ZROWS, zcopy, 0)

    def zwait(k, carry):
        pltpu.make_async_copy(
            zbuf, acc.at[pl.ds(t * ROWS_PER_TILE + k * ZROWS, ZROWS)],
            semz).wait()
        return carry

    lax.fori_loop(0, ROWS_PER_TILE // ZROWS, zwait, 0)

    @pl.when(t == N_SUBCORES - 1)
    def _zero_tail():
        pltpu.sync_copy(zbuf.at[pl.ds(0, TAIL_ROWS)],
                        acc.at[pl.ds(N_SUBCORES * ROWS_PER_TILE, TAIL_ROWS)])


def _edge_phase(xsp, rows_hbm, cols_hbm, vals_hbm, acc,
                colv, rowv, valv, buf0, buf1, c, t, sem0, sem1):
    def super_step(s, carry):
        pltpu.sync_copy(cols_hbm.at[c, t, s], colv)
        pltpu.sync_copy(rows_hbm.at[c, t, s], rowv)
        pltpu.sync_copy(vals_hbm.at[c, t, s], valv)

        pltpu.async_copy(xsp.at[colv.at[0]], buf0, sem0)

        def pair(p, carry2):
            i0 = 2 * p
            pltpu.make_async_copy(xsp.at[colv.at[i0]], buf0, sem0).wait()
            pltpu.async_copy(xsp.at[colv.at[i0 + 1]], buf1, sem1)
            _scale_chunk(buf0, valv, i0)
            pltpu.sync_copy(buf0, acc.at[rowv.at[i0]], add=True)

            pltpu.make_async_copy(xsp.at[colv.at[i0 + 1]], buf1, sem1).wait()

            @pl.when(p < SUBC // 2 - 1)
            def _next_gather():
                pltpu.async_copy(xsp.at[colv.at[i0 + 2]], buf0, sem0)

            _scale_chunk(buf1, valv, i0 + 1)
            pltpu.sync_copy(buf1, acc.at[rowv.at[i0 + 1]], add=True)
            return carry2

        lax.fori_loop(0, SUBC // 2, pair, 0)
        return carry

    lax.fori_loop(0, SUPER, super_step, 0)


def _write_partial(acc, p_hbm, c, t):
    base = t * ROWS_PER_TILE
    pltpu.sync_copy(acc.at[pl.ds(base, ROWS_PER_TILE)],
                    p_hbm.at[c].at[pl.ds(base, ROWS_PER_TILE)])

    @pl.when(t == N_SUBCORES - 1)
    def _out_tail():
        tb = N_SUBCORES * ROWS_PER_TILE
        pltpu.sync_copy(acc.at[pl.ds(tb, TAIL_ROWS)],
                        p_hbm.at[c].at[pl.ds(tb, TAIL_ROWS)])


def _load_x_from_hbm(x_hbm, xsp, t):
    base = t * ROWS_PER_TILE
    pltpu.sync_copy(x_hbm.at[pl.ds(base, ROWS_PER_TILE)],
                    xsp.at[pl.ds(base, ROWS_PER_TILE)])

    @pl.when(t == N_SUBCORES - 1)
    def _tail():
        tb = N_SUBCORES * ROWS_PER_TILE
        pltpu.sync_copy(x_hbm.at[pl.ds(tb, TAIL_ROWS)],
                        xsp.at[pl.ds(tb, TAIL_ROWS)])


def _sum_partials_to_spmem(p_prev, xsp, buf0, buf1, t):
    def sub(k, carry):
        r0 = t * ROWS_PER_TILE + k * SUB
        pltpu.sync_copy(p_prev.at[0].at[pl.ds(r0, SUB)], buf0.at[pl.ds(0, SUB)])
        pltpu.sync_copy(p_prev.at[1].at[pl.ds(r0, SUB)], buf1.at[pl.ds(0, SUB)])
        _add_rows(buf0, buf1, SUB)
        pltpu.sync_copy(buf0.at[pl.ds(0, SUB)], xsp.at[pl.ds(r0, SUB)])
        return carry

    lax.fori_loop(0, ROWS_PER_TILE // SUB, sub, 0)

    @pl.when(t == N_SUBCORES - 1)
    def _tail():
        r0 = N_SUBCORES * ROWS_PER_TILE
        pltpu.sync_copy(p_prev.at[0].at[pl.ds(r0, TAIL_ROWS)],
                        buf0.at[pl.ds(0, TAIL_ROWS)])
        pltpu.sync_copy(p_prev.at[1].at[pl.ds(r0, TAIL_ROWS)],
                        buf1.at[pl.ds(0, TAIL_ROWS)])
        _add_rows(buf0, buf1, TAIL_ROWS)
        pltpu.sync_copy(buf0.at[pl.ds(0, TAIL_ROWS)],
                        xsp.at[pl.ds(r0, TAIL_ROWS)])


def _spmm_first_body(x0_hbm, x1_hbm, rows_hbm, cols_hbm, vals_hbm,
                     p0_hbm, p1_hbm,
                     xsp, acc, colv, rowv, valv, buf0, buf1, zbuf,
                     sem0, sem1, semz):
    c = lax.axis_index("c")
    t = lax.axis_index("s")
    for x_hbm, p_hbm in ((x0_hbm, p0_hbm), (x1_hbm, p1_hbm)):
        _load_x_from_hbm(x_hbm, xsp, t)
        _zero_acc(acc, zbuf, t, semz)
        plsc.subcore_barrier()
        _edge_phase(xsp, rows_hbm, cols_hbm, vals_hbm, acc,
                    colv, rowv, valv, buf0, buf1, c, t, sem0, sem1)
        plsc.subcore_barrier()
        _write_partial(acc, p_hbm, c, t)
        plsc.subcore_barrier()


def _spmm_next_body(pp0_hbm, pp1_hbm, rows_hbm, cols_hbm, vals_hbm,
                    p0_hbm, p1_hbm,
                    xsp, acc, colv, rowv, valv, buf0, buf1, zbuf,
                    sem0, sem1, semz):
    c = lax.axis_index("c")
    t = lax.axis_index("s")
    for pp_hbm, p_hbm in ((pp0_hbm, p0_hbm), (pp1_hbm, p1_hbm)):
        _sum_partials_to_spmem(pp_hbm, xsp, buf0, buf1, t)
        _zero_acc(acc, zbuf, t, semz)
        plsc.subcore_barrier()
        _edge_phase(xsp, rows_hbm, cols_hbm, vals_hbm, acc,
                    colv, rowv, valv, buf0, buf1, c, t, sem0, sem1)
        plsc.subcore_barrier()
        _write_partial(acc, p_hbm, c, t)
        plsc.subcore_barrier()


_MESH = plsc.VectorSubcoreMesh(core_axis_name="c", subcore_axis_name="s",
                               num_cores=N_CORES)
_SCRATCH = [
    pltpu.VMEM_SHARED((N_NODES, HALF), jnp.float32),
    pltpu.VMEM_SHARED((N_NODES, HALF), jnp.float32),
    pltpu.VMEM((SUBC, CHUNK), jnp.int32),
    pltpu.VMEM((SUBC, CHUNK), jnp.int32),
    pltpu.VMEM((SUBC, CHUNK), jnp.float32),
    pltpu.VMEM((CHUNK, HALF), jnp.float32),
    pltpu.VMEM((CHUNK, HALF), jnp.float32),
    pltpu.VMEM((ZROWS, HALF), jnp.float32),
    pltpu.SemaphoreType.DMA,
    pltpu.SemaphoreType.DMA,
    pltpu.SemaphoreType.DMA,
]
_P_TYPE = jax.ShapeDtypeStruct((N_CORES, N_NODES, HALF), jnp.float32)

_spmm_first = pl.kernel(
    _spmm_first_body,
    out_type=(_P_TYPE, _P_TYPE),
    mesh=_MESH,
    scratch_types=_SCRATCH,
    compiler_params=pltpu.CompilerParams(use_tc_tiling_on_sc=False),
)

_spmm_next = pl.kernel(
    _spmm_next_body,
    out_type=(_P_TYPE, _P_TYPE),
    mesh=_MESH,
    scratch_types=_SCRATCH,
    compiler_params=pltpu.CompilerParams(use_tc_tiling_on_sc=False),
)


def _mean_body(e_ref, a0, a1, b0, b1, c0, c1, o_ref):
    for h, p in ((0, (a0, b0, c0)), (1, (a1, b1, c1))):
        sl = pl.ds(h * HALF, HALF)
        o_ref[:, sl] = (e_ref[:, sl]
                        + p[0][0] + p[0][1]
                        + p[1][0] + p[1][1]
                        + p[2][0] + p[2][1]) * 0.25


def _mean4(ego, p10, p11, p20, p21, p30, p31):
    blk = 2000
    grid = N_NODES // blk
    espec = pl.BlockSpec((blk, DIM), lambda i: (i, 0))
    pspec = pl.BlockSpec((N_CORES, blk, HALF), lambda i: (0, i, 0))
    return pl.pallas_call(
        _mean_body,
        grid=(grid,),
        in_specs=[espec, pspec, pspec, pspec, pspec, pspec, pspec],
        out_specs=espec,
        out_shape=jax.ShapeDtypeStruct((N_NODES, DIM), jnp.float32),
    )(ego, p10, p11, p20, p21, p30, p31)


@jax.jit
def kernel(adj_indices, adj_values, user_weight, item_weight):
    shape5 = (N_CORES, N_SUBCORES, SUPER, SUBC, CHUNK)
    pad = N_EDGES_PAD - N_EDGES
    spread = (jnp.arange(pad, dtype=jnp.int32) * 97) % N_NODES
    rows = jnp.concatenate(
        [adj_indices[0].astype(jnp.int32), spread]).reshape(shape5)
    cols = jnp.concatenate(
        [adj_indices[1].astype(jnp.int32), spread]).reshape(shape5)
    vals = jnp.concatenate(
        [adj_values, jnp.zeros((pad,), jnp.float32)]).reshape(shape5)
    ego = jnp.concatenate([user_weight, item_weight], axis=0)
    ego0 = ego[:, :HALF]
    ego1 = ego[:, HALF:]
    p10, p11 = _spmm_first(ego0, ego1, rows, cols, vals)
    p20, p21 = _spmm_next(p10, p11, rows, cols, vals)
    p30, p31 = _spmm_next(p20, p21, rows, cols, vals)
    final = _mean4(ego, p10, p11, p20, p21, p30, p31)
    return final[:N_USERS], final[N_USERS:]


# speedup vs baseline: 3.0381x; 3.0381x over previous
import jax
import jax.numpy as jnp
from jax import lax
from jax.experimental import pallas as pl
from jax.experimental.pallas import tpu as pltpu
from jax.experimental.pallas import tpu_sc as plsc

N_USERS = 6000
N_ITEMS = 4000
N_NODES = N_USERS + N_ITEMS
N_EDGES = 320000
DIM = 128

N_CORES = 2
N_SUBCORES = 16
CHUNK = 128
SUBC = 16
SUPER = 5
EDGES_PER_TILE = SUPER * SUBC * CHUNK
N_EDGES_PAD = N_CORES * N_SUBCORES * EDGES_PER_TILE
ROWS_PER_TILE = 624
TAIL_ROWS = N_NODES - N_SUBCORES * ROWS_PER_TILE
ZROWS = 48


def _scale_chunk(buf, valv, ci):
    def group(g, carry):
        vv = valv[ci, pl.ds(g * 16, 16)]
        for e in range(16):
            idx = jnp.full((16, 1), e, jnp.int32)
            vb = lax.gather(
                vv, idx,
                lax.GatherDimensionNumbers(offset_dims=(),
                                           collapsed_slice_dims=(0,),
                                           start_index_map=(0,)),
                slice_sizes=(1,),
                mode=lax.GatherScatterMode.PROMISE_IN_BOUNDS)
            r = g * 16 + e
            for j in range(DIM // 16):
                buf[r, pl.ds(16 * j, 16)] = buf[r, pl.ds(16 * j, 16)] * vb
        return carry

    lax.fori_loop(0, CHUNK // 16, group, 0)


def _spmm_body(x_hbm, rows_hbm, cols_hbm, vals_hbm, p_hbm,
               acc, colv, rowv, valv, buf0, buf1, zbuf,
               sem0, sem1, semz):
    c = lax.axis_index("c")
    t = lax.axis_index("s")

    zero = jnp.zeros((16,), jnp.float32)
    for r in range(ZROWS):
        for j in range(DIM // 16):
            zbuf[r, pl.ds(16 * j, 16)] = zero

    def zcopy(k, carry):
        pltpu.async_copy(
            zbuf, acc.at[pl.ds(t * ROWS_PER_TILE + k * ZROWS, ZROWS)], semz)
        return carry

    lax.fori_loop(0, ROWS_PER_TILE // ZROWS, zcopy, 0)

    def zwait(k, carry):
        pltpu.make_async_copy(
            zbuf, acc.at[pl.ds(t * ROWS_PER_TILE + k * ZROWS, ZROWS)],
            semz).wait()
        return carry

    lax.fori_loop(0, ROWS_PER_TILE // ZROWS, zwait, 0)

    @pl.when(t == N_SUBCORES - 1)
    def _zero_tail():
        pltpu.sync_copy(zbuf.at[pl.ds(0, TAIL_ROWS)],
                        acc.at[pl.ds(N_SUBCORES * ROWS_PER_TILE, TAIL_ROWS)])

    plsc.subcore_barrier()

    def super_step(s, carry):
        pltpu.sync_copy(cols_hbm.at[c, t, s], colv)
        pltpu.sync_copy(rows_hbm.at[c, t, s], rowv)
        pltpu.sync_copy(vals_hbm.at[c, t, s], valv)

        pltpu.async_copy(x_hbm.at[colv.at[0]], buf0, sem0)

        def pair(p, carry2):
            i0 = 2 * p
            pltpu.make_async_copy(x_hbm.at[colv.at[i0]], buf0, sem0).wait()
            pltpu.async_copy(x_hbm.at[colv.at[i0 + 1]], buf1, sem1)
            _scale_chunk(buf0, valv, i0)
            pltpu.sync_copy(buf0, acc.at[rowv.at[i0]], add=True)

            pltpu.make_async_copy(x_hbm.at[colv.at[i0 + 1]], buf1, sem1).wait()

            @pl.when(p < SUBC // 2 - 1)
            def _next_gather():
                pltpu.async_copy(x_hbm.at[colv.at[i0 + 2]], buf0, sem0)

            _scale_chunk(buf1, valv, i0 + 1)
            pltpu.sync_copy(buf1, acc.at[rowv.at[i0 + 1]], add=True)
            return carry2

        lax.fori_loop(0, SUBC // 2, pair, 0)
        return carry

    lax.fori_loop(0, SUPER, super_step, 0)
    plsc.subcore_barrier()

    base = t * ROWS_PER_TILE
    pltpu.sync_copy(acc.at[pl.ds(base, ROWS_PER_TILE)],
                    p_hbm.at[c].at[pl.ds(base, ROWS_PER_TILE)])

    @pl.when(t == N_SUBCORES - 1)
    def _out_tail():
        tb = N_SUBCORES * ROWS_PER_TILE
        pltpu.sync_copy(acc.at[pl.ds(tb, TAIL_ROWS)],
                        p_hbm.at[c].at[pl.ds(tb, TAIL_ROWS)])


_spmm_sc = pl.kernel(
    _spmm_body,
    out_type=jax.ShapeDtypeStruct((N_CORES, N_NODES, DIM), jnp.float32),
    mesh=plsc.VectorSubcoreMesh(core_axis_name="c", subcore_axis_name="s",
                                num_cores=N_CORES),
    scratch_types=[
        pltpu.VMEM_SHARED((N_NODES, DIM), jnp.float32),
        pltpu.VMEM((SUBC, CHUNK), jnp.int32),
        pltpu.VMEM((SUBC, CHUNK), jnp.int32),
        pltpu.VMEM((SUBC, CHUNK), jnp.float32),
        pltpu.VMEM((CHUNK, DIM), jnp.float32),
        pltpu.VMEM((CHUNK, DIM), jnp.float32),
        pltpu.VMEM((ZROWS, DIM), jnp.float32),
        pltpu.SemaphoreType.DMA,
        pltpu.SemaphoreType.DMA,
        pltpu.SemaphoreType.DMA,
    ],
)

_BLK = 2000
_ESPEC = pl.BlockSpec((_BLK, DIM), lambda i: (i, 0))
_PSPEC = pl.BlockSpec((N_CORES, _BLK, DIM), lambda i: (0, i, 0))


def _sum2_body(p_ref, o_ref):
    o_ref[...] = p_ref[0] + p_ref[1]


def _sum2(p):
    return pl.pallas_call(
        _sum2_body,
        grid=(N_NODES // _BLK,),
        in_specs=[_PSPEC],
        out_specs=_ESPEC,
        out_shape=jax.ShapeDtypeStruct((N_NODES, DIM), jnp.float32),
    )(p)


def _mean_body(e_ref, x1_ref, x2_ref, p3_ref, o_ref):
    o_ref[...] = (e_ref[...] + x1_ref[...] + x2_ref[...]
                  + p3_ref[0] + p3_ref[1]) * 0.25


def _mean4(ego, x1, x2, p3):
    return pl.pallas_call(
        _mean_body,
        grid=(N_NODES // _BLK,),
        in_specs=[_ESPEC, _ESPEC, _ESPEC, _PSPEC],
        out_specs=_ESPEC,
        out_shape=jax.ShapeDtypeStruct((N_NODES, DIM), jnp.float32),
    )(ego, x1, x2, p3)


@jax.jit
def kernel(adj_indices, adj_values, user_weight, item_weight):
    shape5 = (N_CORES, N_SUBCORES, SUPER, SUBC, CHUNK)
    pad = N_EDGES_PAD - N_EDGES
    spread = (jnp.arange(pad, dtype=jnp.int32) * 97) % N_NODES
    rows = jnp.concatenate(
        [adj_indices[0].astype(jnp.int32), spread]).reshape(shape5)
    cols = jnp.concatenate(
        [adj_indices[1].astype(jnp.int32), spread]).reshape(shape5)
    vals = jnp.concatenate(
        [adj_values, jnp.zeros((pad,), jnp.float32)]).reshape(shape5)
    ego = jnp.concatenate([user_weight, item_weight], axis=0)
    p1 = _spmm_sc(ego, rows, cols, vals)
    x1 = _sum2(p1)
    p2 = _spmm_sc(x1, rows, cols, vals)
    x2 = _sum2(p2)
    p3 = _spmm_sc(x2, rows, cols, vals)
    final = _mean4(ego, x1, x2, p3)
    return final[:N_USERS], final[N_USERS:]


# speedup vs baseline: 3.0634x; 1.0083x over previous
import jax
import jax.numpy as jnp
from jax import lax
from jax.experimental import pallas as pl
from jax.experimental.pallas import tpu as pltpu
from jax.experimental.pallas import tpu_sc as plsc

N_USERS = 6000
N_ITEMS = 4000
N_NODES = N_USERS + N_ITEMS
N_EDGES = 320000
DIM = 128

N_CORES = 2
N_SUBCORES = 16
CHUNK = 128
SUBC = 16
SUPER = 5
EDGES_PER_TILE = SUPER * SUBC * CHUNK
N_EDGES_PAD = N_CORES * N_SUBCORES * EDGES_PER_TILE
ROWS_PER_TILE = 624
TAIL_ROWS = N_NODES - N_SUBCORES * ROWS_PER_TILE
ZROWS = 48


def _scale_chunk(buf, valv, ci):
    def group(g, carry):
        vv = valv[ci, pl.ds(g * 16, 16)]
        for e in range(16):
            idx = jnp.full((16, 1), e, jnp.int32)
            vb = lax.gather(
                vv, idx,
                lax.GatherDimensionNumbers(offset_dims=(),
                                           collapsed_slice_dims=(0,),
                                           start_index_map=(0,)),
                slice_sizes=(1,),
                mode=lax.GatherScatterMode.PROMISE_IN_BOUNDS)
            r = g * 16 + e
            for j in range(DIM // 16):
                buf[r, pl.ds(16 * j, 16)] = buf[r, pl.ds(16 * j, 16)] * vb
        return carry

    lax.fori_loop(0, CHUNK // 16, group, 0)


def _spmm_body(x_hbm, rows_hbm, cols_hbm, vals_hbm, p_hbm,
               acc, colv, rowv, valv, buf0, buf1, zbuf,
               sem0, sem0b, sem1, sem1b, semz):
    c = lax.axis_index("c")
    t = lax.axis_index("s")

    zero = jnp.zeros((16,), jnp.float32)
    for r in range(ZROWS):
        for j in range(DIM // 16):
            zbuf[r, pl.ds(16 * j, 16)] = zero

    def zcopy(k, carry):
        pltpu.async_copy(
            zbuf, acc.at[pl.ds(t * ROWS_PER_TILE + k * ZROWS, ZROWS)], semz)
        return carry

    lax.fori_loop(0, ROWS_PER_TILE // ZROWS, zcopy, 0)

    def zwait(k, carry):
        pltpu.make_async_copy(
            zbuf, acc.at[pl.ds(t * ROWS_PER_TILE + k * ZROWS, ZROWS)],
            semz).wait()
        return carry

    lax.fori_loop(0, ROWS_PER_TILE // ZROWS, zwait, 0)

    @pl.when(t == N_SUBCORES - 1)
    def _zero_tail():
        pltpu.sync_copy(zbuf.at[pl.ds(0, TAIL_ROWS)],
                        acc.at[pl.ds(N_SUBCORES * ROWS_PER_TILE, TAIL_ROWS)])

    plsc.subcore_barrier()

    HC = CHUNK // 2

    def _gather2(ci, buf, sa, sb):
        pltpu.async_copy(x_hbm.at[colv.at[ci, pl.ds(0, HC)]],
                         buf.at[pl.ds(0, HC)], sa)
        pltpu.async_copy(x_hbm.at[colv.at[ci, pl.ds(HC, HC)]],
                         buf.at[pl.ds(HC, HC)], sb)

    def _gather2_wait(ci, buf, sa, sb):
        pltpu.make_async_copy(x_hbm.at[colv.at[ci, pl.ds(0, HC)]],
                              buf.at[pl.ds(0, HC)], sa).wait()
        pltpu.make_async_copy(x_hbm.at[colv.at[ci, pl.ds(HC, HC)]],
                              buf.at[pl.ds(HC, HC)], sb).wait()

    def super_step(s, carry):
        pltpu.sync_copy(cols_hbm.at[c, t, s], colv)
        pltpu.sync_copy(rows_hbm.at[c, t, s], rowv)
        pltpu.sync_copy(vals_hbm.at[c, t, s], valv)

        _gather2(0, buf0, sem0, sem0b)

        def pair(p, carry2):
            i0 = 2 * p
            _gather2_wait(i0, buf0, sem0, sem0b)
            _gather2(i0 + 1, buf1, sem1, sem1b)
            _scale_chunk(buf0, valv, i0)
            pltpu.sync_copy(buf0, acc.at[rowv.at[i0]], add=True)

            _gather2_wait(i0 + 1, buf1, sem1, sem1b)

            @pl.when(p < SUBC // 2 - 1)
            def _next_gather():
                _gather2(i0 + 2, buf0, sem0, sem0b)

            _scale_chunk(buf1, valv, i0 + 1)
            pltpu.sync_copy(buf1, acc.at[rowv.at[i0 + 1]], add=True)
            return carry2

        lax.fori_loop(0, SUBC // 2, pair, 0)
        return carry

    lax.fori_loop(0, SUPER, super_step, 0)
    plsc.subcore_barrier()

    base = t * ROWS_PER_TILE
    pltpu.sync_copy(acc.at[pl.ds(base, ROWS_PER_TILE)],
                    p_hbm.at[c].at[pl.ds(base, ROWS_PER_TILE)])

    @pl.when(t == N_SUBCORES - 1)
    def _out_tail():
        tb = N_SUBCORES * ROWS_PER_TILE
        pltpu.sync_copy(acc.at[pl.ds(tb, TAIL_ROWS)],
                        p_hbm.at[c].at[pl.ds(tb, TAIL_ROWS)])


_spmm_sc = pl.kernel(
    _spmm_body,
    out_type=jax.ShapeDtypeStruct((N_CORES, N_NODES, DIM), jnp.float32),
    mesh=plsc.VectorSubcoreMesh(core_axis_name="c", subcore_axis_name="s",
                                num_cores=N_CORES),
    scratch_types=[
        pltpu.VMEM_SHARED((N_NODES, DIM), jnp.float32),
        pltpu.VMEM((SUBC, CHUNK), jnp.int32),
        pltpu.VMEM((SUBC, CHUNK), jnp.int32),
        pltpu.VMEM((SUBC, CHUNK), jnp.float32),
        pltpu.VMEM((CHUNK, DIM), jnp.float32),
        pltpu.VMEM((CHUNK, DIM), jnp.float32),
        pltpu.VMEM((ZROWS, DIM), jnp.float32),
        pltpu.SemaphoreType.DMA,
        pltpu.SemaphoreType.DMA,
        pltpu.SemaphoreType.DMA,
        pltpu.SemaphoreType.DMA,
        pltpu.SemaphoreType.DMA,
    ],
)

_BLK = 2000
_ESPEC = pl.BlockSpec((_BLK, DIM), lambda i: (i, 0))
_PSPEC = pl.BlockSpec((N_CORES, _BLK, DIM), lambda i: (0, i, 0))


def _sum2_body(p_ref, o_ref):
    o_ref[...] = p_ref[0] + p_ref[1]


def _sum2(p):
    return pl.pallas_call(
        _sum2_body,
        grid=(N_NODES // _BLK,),
        in_specs=[_PSPEC],
        out_specs=_ESPEC,
        out_shape=jax.ShapeDtypeStruct((N_NODES, DIM), jnp.float32),
    )(p)


def _mean_body(e_ref, x1_ref, x2_ref, p3_ref, o_ref):
    o_ref[...] = (e_ref[...] + x1_ref[...] + x2_ref[...]
                  + p3_ref[0] + p3_ref[1]) * 0.25


def _mean4(ego, x1, x2, p3):
    return pl.pallas_call(
        _mean_body,
        grid=(N_NODES // _BLK,),
        in_specs=[_ESPEC, _ESPEC, _ESPEC, _PSPEC],
        out_specs=_ESPEC,
        out_shape=jax.ShapeDtypeStruct((N_NODES, DIM), jnp.float32),
    )(ego, x1, x2, p3)


@jax.jit
def kernel(adj_indices, adj_values, user_weight, item_weight):
    shape5 = (N_CORES, N_SUBCORES, SUPER, SUBC, CHUNK)
    pad = N_EDGES_PAD - N_EDGES
    spread = (jnp.arange(pad, dtype=jnp.int32) * 97) % N_NODES
    rows = jnp.concatenate(
        [adj_indices[0].astype(jnp.int32), spread]).reshape(shape5)
    cols = jnp.concatenate(
        [adj_indices[1].astype(jnp.int32), spread]).reshape(shape5)
    vals = jnp.concatenate(
        [adj_values, jnp.zeros((pad,), jnp.float32)]).reshape(shape5)
    ego = jnp.concatenate([user_weight, item_weight], axis=0)
    p1 = _spmm_sc(ego, rows, cols, vals)
    x1 = _sum2(p1)
    p2 = _spmm_sc(x1, rows, cols, vals)
    x2 = _sum2(p2)
    p3 = _spmm_sc(x2, rows, cols, vals)
    final = _mean4(ego, x1, x2, p3)
    return final[:N_USERS], final[N_USERS:]
